# baseline (speedup 1.0000x reference)
import functools

import jax
import jax.numpy as jnp
from jax import lax
from jax.experimental import pallas as pl
from jax.experimental.pallas import tpu as pltpu
from jax.experimental.pallas import tpu_sc as plsc

NC = 2
NS = 16
NW = NC * NS
L = 16
C = 128
B = 4

_mesh = lambda: plsc.VectorSubcoreMesh(core_axis_name="c", subcore_axis_name="s",
                                       num_cores=NC, num_subcores=NS)
_SC_PARAMS = pltpu.CompilerParams(needs_layout_passes=False,
                                  use_tc_tiling_on_sc=False)


def _wid():
    return lax.axis_index("s") * NC + lax.axis_index("c")


def _sc_deg(dst3, ew3, n_nodes):
    nch = dst3.shape[1]

    @functools.partial(
        pl.kernel,
        out_type=jax.ShapeDtypeStruct((NW, n_nodes), jnp.float32),
        mesh=_mesh(),
        compiler_params=_SC_PARAMS,
        scratch_types=[
            pltpu.VMEM((n_nodes,), jnp.float32),
            pltpu.VMEM((nch, C), jnp.int32),
            pltpu.VMEM((nch, C), jnp.float32),
        ],
    )
    def k(dst_hbm, ew_hbm, out_hbm, hist, dstq, ewq):
        wid = _wid()
        zero = jnp.zeros((L,), jnp.float32)
        mask0 = jnp.arange(L, dtype=jnp.int32) == 0

        @pl.loop(0, n_nodes, step=L)
        def _(j):
            hist[pl.ds(j, L)] = zero

        pltpu.sync_copy(dst_hbm.at[wid], dstq)
        pltpu.sync_copy(ew_hbm.at[wid], ewq)

        @pl.loop(0, nch)
        def _(j):
            jj = jnp.full((L,), j, jnp.int32)

            @pl.loop(0, C)
            def _(i):
                ii = jnp.full((L,), i, jnp.int32)
                d = plsc.load_gather(dstq, [jj, ii])
                w = plsc.load_gather(ewq, [jj, ii])
                plsc.addupdate_scatter(hist, [d], w, mask=mask0)

        pltpu.sync_copy(hist, out_hbm.at[wid])

    return k(dst3, ew3)


def _sc_agg(feat_h, src3, dst3, ewn3, dinv, n_nodes, compute_norm):
    d2 = feat_h.shape[2]
    nch = src3.shape[1]
    npw = n_nodes // NS
    nzf = npw // C
    nzr = npw - nzf * C
    bb = 2 * B
    assert nch % bb == 0

    out_type = [jax.ShapeDtypeStruct((NC, 2, n_nodes, d2), jnp.float32)]
    if compute_norm:
        out_type.append(jax.ShapeDtypeStruct((NW, nch, C), jnp.float32))

    scratch = [
        pltpu.VMEM_SHARED((n_nodes, d2), jnp.float32),
        pltpu.VMEM_SHARED((n_nodes, d2), jnp.float32),
        pltpu.VMEM((C, d2), jnp.float32),
        pltpu.VMEM((C, d2), jnp.float32),
        pltpu.VMEM((B, C), jnp.int32),
        pltpu.VMEM((B, C), jnp.int32),
        pltpu.VMEM((B, C), jnp.int32),
        pltpu.VMEM((B, C), jnp.int32),
        pltpu.VMEM((B, C), jnp.float32),
        pltpu.VMEM((B, C), jnp.float32),
        pltpu.SemaphoreType.DMA,
        pltpu.SemaphoreType.DMA,
        pltpu.SemaphoreType.DMA,
        pltpu.SemaphoreType.DMA,
        pltpu.SemaphoreType.DMA,
        pltpu.SemaphoreType.DMA,
    ]
    if compute_norm:
        scratch += [
            pltpu.VMEM((B, C), jnp.float32),
            pltpu.VMEM((B, C), jnp.float32),
            pltpu.SemaphoreType.DMA,
            pltpu.SemaphoreType.DMA,
            pltpu.VMEM((n_nodes,), jnp.float32),
        ]

    @functools.partial(pl.kernel, out_type=out_type, mesh=_mesh(),
                       scratch_types=scratch, compiler_params=_SC_PARAMS)
    def k(feat_hbm, src_hbm, dst_hbm, ewn_hbm, dinv_hbm, *refs):
        if compute_norm:
            (p_hbm, norm_hbm, fsp, acc, rows0, rows1, srcA, srcB, dstA, dstB,
             wnA, wnB, g0, g1, s0, s1, ib0, ib1, noA, noB, na0, na1,
             dinv_v) = refs
        else:
            (p_hbm, fsp, acc, rows0, rows1, srcA, srcB, dstA, dstB,
             wnA, wnB, g0, g1, s0, s1, ib0, ib1) = refs
            norm_hbm = ewn_hbm
        core = lax.axis_index("c")
        sub = lax.axis_index("s")
        wid = sub * NC + core
        zero = jnp.zeros((L,), jnp.float32)
        rows = (rows0, rows1)
        gsem = (g0, g1)
        ssem = (s0, s1)
        srcb = (srcA, srcB)
        dstb = (dstA, dstB)
        wnb = (wnA, wnB)

        if compute_norm:
            pltpu.sync_copy(dinv_hbm, dinv_v)
            nob = (noA, noB)
            nasem = (na0, na1)

        for h in range(2):
            derive_norm = compute_norm and h == 0
            wsrc_hbm = ewn_hbm if derive_norm else norm_hbm

            pltpu.sync_copy(feat_hbm.at[h, pl.ds(sub * npw, npw), :],
                            fsp.at[pl.ds(sub * npw, npw), :])

            @pl.loop(0, C)
            def _(i):
                for g in range(d2 // L):
                    rows0[i, pl.ds(g * L, L)] = zero

            @pl.loop(0, nzf)
            def _(t):
                pltpu.sync_copy(rows0, acc.at[pl.ds(sub * npw + t * C, C), :])

            if nzr:
                pltpu.sync_copy(rows0.at[pl.ds(0, nzr)],
                                acc.at[pl.ds(sub * npw + nzf * C, nzr), :])

            plsc.subcore_barrier()

            def idx_block(half, j0, sem):
                for hbm, buf in ((src_hbm, srcb[half]), (dst_hbm, dstb[half]),
                                 (wsrc_hbm, wnb[half])):
                    pltpu.async_copy(hbm.at[wid, pl.ds(j0, B), :], buf, sem)

            def wait_idx_block(half, j0, sem):
                for hbm, buf in ((src_hbm, srcb[half]), (dst_hbm, dstb[half]),
                                 (wsrc_hbm, wnb[half])):
                    pltpu.make_async_copy(hbm.at[wid, pl.ds(j0, B), :], buf,
                                          sem).wait()

            def scale(buf, half, r):
                nsrc = nob[half] if derive_norm else wnb[half]
                if derive_norm:
                    for g in range(C // L):
                        sl = pl.ds(g * L, L)
                        nv = (plsc.load_gather(dinv_v, [srcb[half][r, sl]])
                              * wnb[half][r, sl]
                              * plsc.load_gather(dinv_v, [dstb[half][r, sl]]))
                        nob[half][r, sl] = nv

                rsp = jnp.full((L,), r, jnp.int32)

                @pl.loop(0, C)
                def _(i):
                    nsp = plsc.load_gather(
                        nsrc, [rsp, jnp.full((L,), i, jnp.int32)])
                    for g in range(d2 // L):
                        sl = pl.ds(g * L, L)
                        buf[i, sl] = buf[i, sl] * nsp

            idx_block(0, 0, ib0)
            idx_block(1, B, ib1)
            wait_idx_block(0, 0, ib0)

            @pl.loop(0, nch, step=bb)
            def _(j0):
                more = j0 + bb < nch
                pltpu.async_copy(fsp.at[srcA.at[0]], rows0, g0)
                pltpu.async_copy(fsp.at[srcA.at[1]], rows1, g1)
                for c in range(bb):
                    p = c % 2
                    half = c // B
                    r = c % B
                    if c == B - 2:
                        wait_idx_block(1, j0 + B, ib1)
                    if c == B:
                        @pl.when(more)
                        def _():
                            idx_block(0, j0 + bb, ib0)

                    pltpu.make_async_copy(
                        fsp.at[srcb[half].at[r]], rows[p], gsem[p]).wait()
                    scale(rows[p], half, r)
                    sd = pltpu.async_copy(rows[p], acc.at[dstb[half].at[r]],
                                          ssem[p], add=True)
                    if c < bb - 2:
                        sd.wait()
                        c2 = c + 2
                        pltpu.async_copy(
                            fsp.at[srcb[c2 // B].at[c2 % B]], rows[p],
                            gsem[p])
                    if derive_norm and r == B - 1:
                        pltpu.async_copy(
                            nob[half],
                            norm_hbm.at[wid, pl.ds(j0 + half * B, B), :],
                            nasem[half])

                pltpu.make_async_copy(rows[0], acc.at[dstb[1].at[B - 2]],
                                      ssem[0]).wait()
                pltpu.make_async_copy(rows[1], acc.at[dstb[1].at[B - 1]],
                                      ssem[1]).wait()
                if derive_norm:
                    for half in range(2):
                        pltpu.make_async_copy(
                            nob[half],
                            norm_hbm.at[wid, pl.ds(j0 + half * B, B), :],
                            nasem[half]).wait()

                @pl.when(more)
                def _():
                    idx_block(1, j0 + bb + B, ib1)
                    wait_idx_block(0, j0 + bb, ib0)

            plsc.subcore_barrier()

            pltpu.sync_copy(acc.at[pl.ds(sub * npw, npw), :],
                            p_hbm.at[core, h, pl.ds(sub * npw, npw), :])

            if h == 0:
                plsc.subcore_barrier()

    return k(feat_h, src3, dst3, ewn3, dinv)


def _tc_deg_finish(deg_parts):
    def body(dp_ref, dinv_ref, dinv2_ref):
        deg = jnp.sum(dp_ref[...], axis=0) + 1.0
        dinv = jnp.where(deg > 0, lax.rsqrt(jnp.maximum(deg, 1e-12)), 0.0)
        dinv_ref[...] = dinv
        dinv2_ref[...] = dinv * dinv

    n = deg_parts.shape[1]
    return pl.pallas_call(
        body,
        out_shape=[jax.ShapeDtypeStruct((n,), jnp.float32),
                   jax.ShapeDtypeStruct((n,), jnp.float32)],
    )(deg_parts)


def _tc_layer1(p, x_h, dinv2_col, w_q, b_h, d2_out):
    n = x_h.shape[1]

    def body(p_ref, f_ref, d2_ref, w00, w01, w10, w11, b_ref, o_ref):
        agg0 = p_ref[0, 0] + p_ref[1, 0] + f_ref[0] * d2_ref[...]
        agg1 = p_ref[0, 1] + p_ref[1, 1] + f_ref[1] * d2_ref[...]
        w = ((w00, w01), (w10, w11))
        for o in range(2):
            y = jnp.dot(agg0, w[0][o][...], preferred_element_type=jnp.float32)
            y = y + jnp.dot(agg1, w[1][o][...],
                            preferred_element_type=jnp.float32)
            y = y + b_ref[o]
            o_ref[o] = jnp.maximum(y, 0.0)

    bn = 2000
    d2 = x_h.shape[2]
    full = lambda *s: pl.BlockSpec(s, lambda i: (0,) * len(s))
    return pl.pallas_call(
        body,
        grid=(n // bn,),
        in_specs=[
            pl.BlockSpec((NC, 2, bn, d2), lambda i: (0, 0, i, 0)),
            pl.BlockSpec((2, bn, d2), lambda i: (0, i, 0)),
            pl.BlockSpec((bn, 1), lambda i: (i, 0)),
            full(d2, d2_out), full(d2, d2_out), full(d2, d2_out),
            full(d2, d2_out), full(2, d2_out),
        ],
        out_specs=pl.BlockSpec((2, bn, d2_out), lambda i: (0, i, 0)),
        out_shape=jax.ShapeDtypeStruct((2, n, d2_out), jnp.float32),
    )(p, x_h, dinv2_col, w_q[0][0], w_q[0][1], w_q[1][0], w_q[1][1], b_h)


def _tc_layer2(p, h_h, dinv2_col, w_halves, b):
    n = h_h.shape[1]
    dout = w_halves[0].shape[1]

    def body(p_ref, f_ref, d2_ref, wa_ref, wb_ref, b_ref, o_ref):
        agg0 = p_ref[0, 0] + p_ref[1, 0] + f_ref[0] * d2_ref[...]
        agg1 = p_ref[0, 1] + p_ref[1, 1] + f_ref[1] * d2_ref[...]
        y = jnp.dot(agg0, wa_ref[...], preferred_element_type=jnp.float32)
        y = y + jnp.dot(agg1, wb_ref[...], preferred_element_type=jnp.float32)
        o_ref[...] = y + b_ref[...]

    bn = 2000
    d2 = h_h.shape[2]
    full = lambda *s: pl.BlockSpec(s, lambda i: (0,) * len(s))
    return pl.pallas_call(
        body,
        grid=(n // bn,),
        in_specs=[
            pl.BlockSpec((NC, 2, bn, d2), lambda i: (0, 0, i, 0)),
            pl.BlockSpec((2, bn, d2), lambda i: (0, i, 0)),
            pl.BlockSpec((bn, 1), lambda i: (i, 0)),
            full(d2, dout), full(d2, dout), full(dout),
        ],
        out_specs=pl.BlockSpec((bn, dout), lambda i: (i, 0)),
        out_shape=jax.ShapeDtypeStruct((n, dout), jnp.float32),
    )(p, h_h, dinv2_col, w_halves[0], w_halves[1], b)


def kernel(x, edge_index, edge_weight, W1, b1, W2, b2):
    n, d_in = x.shape
    e = edge_weight.shape[0]
    d_h = W1.shape[1]
    d_h_pad = ((d_h + 2 * L - 1) // (2 * L)) * (2 * L)
    da = d_in // 2
    db = d_h_pad // 2

    W1p = jnp.zeros((d_in, d_h_pad), jnp.float32).at[:, :d_h].set(W1)
    b1p = jnp.zeros((d_h_pad,), jnp.float32).at[:d_h].set(b1)
    W2p = jnp.zeros((d_h_pad, W2.shape[1]), jnp.float32).at[:d_h, :].set(W2)

    w1_q = ((W1p[:da, :db], W1p[:da, db:]),
            (W1p[da:, :db], W1p[da:, db:]))
    b1_h = jnp.stack([b1p[:db], b1p[db:]])
    w2_h = (W2p[:db], W2p[db:])

    ew_tile = e // NW
    nch = -(-ew_tile // (2 * B * C)) * 2 * B
    pad = nch * C - ew_tile
    src3 = jnp.pad(edge_index[0].reshape(NW, ew_tile), ((0, 0), (0, pad)),
                   constant_values=0).reshape(NW, nch, C)
    dst3 = jnp.pad(edge_index[1].reshape(NW, ew_tile), ((0, 0), (0, pad)),
                   constant_values=0).reshape(NW, nch, C)
    ew3 = jnp.pad(edge_weight.reshape(NW, ew_tile), ((0, 0), (0, pad)),
                  constant_values=0.0).reshape(NW, nch, C)

    x_h = jnp.stack([x[:, :da], x[:, da:]])

    deg_parts = _sc_deg(dst3, ew3, n)
    dinv, dinv2 = _tc_deg_finish(deg_parts)
    dinv2_col = dinv2.reshape(n, 1)

    p1, norm3 = _sc_agg(x_h, src3, dst3, ew3, dinv, n, compute_norm=True)
    h_h = _tc_layer1(p1, x_h, dinv2_col, w1_q, b1_h, db)
    (p2,) = _sc_agg(h_h, src3, dst3, norm3, dinv, n, compute_norm=False)
    out = _tc_layer2(p2, h_h, dinv2_col, w2_h, b2)
    return out


# speedup vs baseline: 1.0163x; 1.0163x over previous
import functools

import jax
import jax.numpy as jnp
from jax import lax
from jax.experimental import pallas as pl
from jax.experimental.pallas import tpu as pltpu
from jax.experimental.pallas import tpu_sc as plsc

NC = 2
NS = 16
NW = NC * NS
L = 16
C = 128
B = 4

_mesh = lambda: plsc.VectorSubcoreMesh(core_axis_name="c", subcore_axis_name="s",
                                       num_cores=NC, num_subcores=NS)
_SC_PARAMS = pltpu.CompilerParams(needs_layout_passes=False,
                                  use_tc_tiling_on_sc=False)


def _wid():
    return lax.axis_index("s") * NC + lax.axis_index("c")


def _sc_deg(dst3, ew3, n_nodes):
    nch = dst3.shape[1]

    @functools.partial(
        pl.kernel,
        out_type=jax.ShapeDtypeStruct((NW, n_nodes), jnp.float32),
        mesh=_mesh(),
        compiler_params=_SC_PARAMS,
        scratch_types=[
            pltpu.VMEM((n_nodes,), jnp.float32),
            pltpu.VMEM((nch, C), jnp.int32),
            pltpu.VMEM((nch, C), jnp.float32),
        ],
    )
    def k(dst_hbm, ew_hbm, out_hbm, hist, dstq, ewq):
        wid = _wid()
        zero = jnp.zeros((L,), jnp.float32)
        mask0 = jnp.arange(L, dtype=jnp.int32) == 0

        @pl.loop(0, n_nodes, step=L)
        def _(j):
            hist[pl.ds(j, L)] = zero

        pltpu.sync_copy(dst_hbm.at[wid], dstq)
        pltpu.sync_copy(ew_hbm.at[wid], ewq)

        @pl.loop(0, nch)
        def _(j):
            jj = jnp.full((L,), j, jnp.int32)

            @pl.loop(0, C)
            def _(i):
                ii = jnp.full((L,), i, jnp.int32)
                d = plsc.load_gather(dstq, [jj, ii])
                w = plsc.load_gather(ewq, [jj, ii])
                plsc.addupdate_scatter(hist, [d], w, mask=mask0)

        pltpu.sync_copy(hist, out_hbm.at[wid])

    return k(dst3, ew3)


def _sc_agg(feat_h, src3, dst3, ewn3, dinv, n_nodes, compute_norm):
    d2 = feat_h.shape[2]
    nch = src3.shape[1]
    npw = n_nodes // NS
    nzf = npw // C
    nzr = npw - nzf * C
    bb = 2 * B
    assert nch % bb == 0

    out_type = [jax.ShapeDtypeStruct((NC, 2, n_nodes, d2), jnp.float32)]
    if compute_norm:
        out_type.append(jax.ShapeDtypeStruct((NW, nch, C), jnp.float32))

    scratch = [
        pltpu.VMEM_SHARED((n_nodes, d2), jnp.float32),
        pltpu.VMEM_SHARED((n_nodes, d2), jnp.float32),
        pltpu.VMEM((C, d2), jnp.float32),
        pltpu.VMEM((C, d2), jnp.float32),
        pltpu.VMEM((B, C), jnp.int32),
        pltpu.VMEM((B, C), jnp.int32),
        pltpu.VMEM((B, C), jnp.int32),
        pltpu.VMEM((B, C), jnp.int32),
        pltpu.VMEM((B, C), jnp.float32),
        pltpu.VMEM((B, C), jnp.float32),
        pltpu.SemaphoreType.DMA,
        pltpu.SemaphoreType.DMA,
        pltpu.SemaphoreType.DMA,
        pltpu.SemaphoreType.DMA,
        pltpu.SemaphoreType.DMA,
        pltpu.SemaphoreType.DMA,
    ]
    if compute_norm:
        scratch += [
            pltpu.VMEM((B, C), jnp.float32),
            pltpu.VMEM((B, C), jnp.float32),
            pltpu.SemaphoreType.DMA,
            pltpu.SemaphoreType.DMA,
            pltpu.VMEM((n_nodes,), jnp.float32),
        ]

    @functools.partial(pl.kernel, out_type=out_type, mesh=_mesh(),
                       scratch_types=scratch, compiler_params=_SC_PARAMS)
    def k(feat_hbm, src_hbm, dst_hbm, ewn_hbm, dinv_hbm, *refs):
        if compute_norm:
            (p_hbm, norm_hbm, fsp, acc, rows0, rows1, srcA, srcB, dstA, dstB,
             wnA, wnB, g0, g1, s0, s1, ib0, ib1, noA, noB, na0, na1,
             dinv_v) = refs
        else:
            (p_hbm, fsp, acc, rows0, rows1, srcA, srcB, dstA, dstB,
             wnA, wnB, g0, g1, s0, s1, ib0, ib1) = refs
            norm_hbm = ewn_hbm
        core = lax.axis_index("c")
        sub = lax.axis_index("s")
        wid = sub * NC + core
        zero = jnp.zeros((L,), jnp.float32)
        rows = (rows0, rows1)
        gsem = (g0, g1)
        ssem = (s0, s1)
        srcb = (srcA, srcB)
        dstb = (dstA, dstB)
        wnb = (wnA, wnB)

        if compute_norm:
            pltpu.sync_copy(dinv_hbm, dinv_v)
            nob = (noA, noB)
            nasem = (na0, na1)

        for h in range(2):
            derive_norm = compute_norm and h == 0
            wsrc_hbm = ewn_hbm if derive_norm else norm_hbm

            pltpu.sync_copy(feat_hbm.at[h, pl.ds(sub * npw, npw), :],
                            fsp.at[pl.ds(sub * npw, npw), :])

            @pl.loop(0, C)
            def _(i):
                for g in range(d2 // L):
                    rows0[i, pl.ds(g * L, L)] = zero

            @pl.loop(0, nzf)
            def _(t):
                pltpu.sync_copy(rows0, acc.at[pl.ds(sub * npw + t * C, C), :])

            if nzr:
                pltpu.sync_copy(rows0.at[pl.ds(0, nzr)],
                                acc.at[pl.ds(sub * npw + nzf * C, nzr), :])

            plsc.subcore_barrier()

            def idx_block(half, j0, sem):
                for hbm, buf in ((src_hbm, srcb[half]), (dst_hbm, dstb[half]),
                                 (wsrc_hbm, wnb[half])):
                    pltpu.async_copy(hbm.at[wid, pl.ds(j0, B), :], buf, sem)

            def wait_idx_block(half, j0, sem):
                for hbm, buf in ((src_hbm, srcb[half]), (dst_hbm, dstb[half]),
                                 (wsrc_hbm, wnb[half])):
                    pltpu.make_async_copy(hbm.at[wid, pl.ds(j0, B), :], buf,
                                          sem).wait()

            def scale(buf, half, r):
                nsrc = nob[half] if derive_norm else wnb[half]
                if derive_norm:
                    for g in range(C // L):
                        sl = pl.ds(g * L, L)
                        nv = (plsc.load_gather(dinv_v, [srcb[half][r, sl]])
                              * wnb[half][r, sl]
                              * plsc.load_gather(dinv_v, [dstb[half][r, sl]]))
                        nob[half][r, sl] = nv

                rsp = jnp.full((L,), r, jnp.int32)

                @pl.loop(0, C, unroll=4)
                def _(i):
                    nsp = plsc.load_gather(
                        nsrc, [rsp, jnp.full((L,), i, jnp.int32)])
                    for g in range(d2 // L):
                        sl = pl.ds(g * L, L)
                        buf[i, sl] = buf[i, sl] * nsp

            idx_block(0, 0, ib0)
            idx_block(1, B, ib1)
            wait_idx_block(0, 0, ib0)

            @pl.loop(0, nch, step=bb)
            def _(j0):
                more = j0 + bb < nch
                pltpu.async_copy(fsp.at[srcA.at[0]], rows0, g0)
                pltpu.async_copy(fsp.at[srcA.at[1]], rows1, g1)
                for c in range(bb):
                    p = c % 2
                    half = c // B
                    r = c % B
                    if c == B - 2:
                        wait_idx_block(1, j0 + B, ib1)
                    if c == B:
                        @pl.when(more)
                        def _():
                            idx_block(0, j0 + bb, ib0)

                    pltpu.make_async_copy(
                        fsp.at[srcb[half].at[r]], rows[p], gsem[p]).wait()
                    scale(rows[p], half, r)
                    sd = pltpu.async_copy(rows[p], acc.at[dstb[half].at[r]],
                                          ssem[p], add=True)
                    if c < bb - 2:
                        sd.wait()
                        c2 = c + 2
                        pltpu.async_copy(
                            fsp.at[srcb[c2 // B].at[c2 % B]], rows[p],
                            gsem[p])
                    if derive_norm and r == B - 1:
                        pltpu.async_copy(
                            nob[half],
                            norm_hbm.at[wid, pl.ds(j0 + half * B, B), :],
                            nasem[half])

                pltpu.make_async_copy(rows[0], acc.at[dstb[1].at[B - 2]],
                                      ssem[0]).wait()
                pltpu.make_async_copy(rows[1], acc.at[dstb[1].at[B - 1]],
                                      ssem[1]).wait()
                if derive_norm:
                    for half in range(2):
                        pltpu.make_async_copy(
                            nob[half],
                            norm_hbm.at[wid, pl.ds(j0 + half * B, B), :],
                            nasem[half]).wait()

                @pl.when(more)
                def _():
                    idx_block(1, j0 + bb + B, ib1)
                    wait_idx_block(0, j0 + bb, ib0)

            plsc.subcore_barrier()

            pltpu.sync_copy(acc.at[pl.ds(sub * npw, npw), :],
                            p_hbm.at[core, h, pl.ds(sub * npw, npw), :])

            if h == 0:
                plsc.subcore_barrier()

    return k(feat_h, src3, dst3, ewn3, dinv)


def _tc_deg_finish(deg_parts):
    def body(dp_ref, dinv_ref, dinv2_ref):
        deg = jnp.sum(dp_ref[...], axis=0) + 1.0
        dinv = jnp.where(deg > 0, lax.rsqrt(jnp.maximum(deg, 1e-12)), 0.0)
        dinv_ref[...] = dinv
        dinv2_ref[...] = dinv * dinv

    n = deg_parts.shape[1]
    return pl.pallas_call(
        body,
        out_shape=[jax.ShapeDtypeStruct((n,), jnp.float32),
                   jax.ShapeDtypeStruct((n,), jnp.float32)],
    )(deg_parts)


def _tc_layer1(p, x_h, dinv2_col, w_q, b_h, d2_out):
    n = x_h.shape[1]

    def body(p_ref, f_ref, d2_ref, w00, w01, w10, w11, b_ref, o_ref):
        agg0 = p_ref[0, 0] + p_ref[1, 0] + f_ref[0] * d2_ref[...]
        agg1 = p_ref[0, 1] + p_ref[1, 1] + f_ref[1] * d2_ref[...]
        w = ((w00, w01), (w10, w11))
        for o in range(2):
            y = jnp.dot(agg0, w[0][o][...], preferred_element_type=jnp.float32)
            y = y + jnp.dot(agg1, w[1][o][...],
                            preferred_element_type=jnp.float32)
            y = y + b_ref[o]
            o_ref[o] = jnp.maximum(y, 0.0)

    bn = 2000
    d2 = x_h.shape[2]
    full = lambda *s: pl.BlockSpec(s, lambda i: (0,) * len(s))
    return pl.pallas_call(
        body,
        grid=(n // bn,),
        in_specs=[
            pl.BlockSpec((NC, 2, bn, d2), lambda i: (0, 0, i, 0)),
            pl.BlockSpec((2, bn, d2), lambda i: (0, i, 0)),
            pl.BlockSpec((bn, 1), lambda i: (i, 0)),
            full(d2, d2_out), full(d2, d2_out), full(d2, d2_out),
            full(d2, d2_out), full(2, d2_out),
        ],
        out_specs=pl.BlockSpec((2, bn, d2_out), lambda i: (0, i, 0)),
        out_shape=jax.ShapeDtypeStruct((2, n, d2_out), jnp.float32),
    )(p, x_h, dinv2_col, w_q[0][0], w_q[0][1], w_q[1][0], w_q[1][1], b_h)


def _tc_layer2(p, h_h, dinv2_col, w_halves, b):
    n = h_h.shape[1]
    dout = w_halves[0].shape[1]

    def body(p_ref, f_ref, d2_ref, wa_ref, wb_ref, b_ref, o_ref):
        agg0 = p_ref[0, 0] + p_ref[1, 0] + f_ref[0] * d2_ref[...]
        agg1 = p_ref[0, 1] + p_ref[1, 1] + f_ref[1] * d2_ref[...]
        y = jnp.dot(agg0, wa_ref[...], preferred_element_type=jnp.float32)
        y = y + jnp.dot(agg1, wb_ref[...], preferred_element_type=jnp.float32)
        o_ref[...] = y + b_ref[...]

    bn = 2000
    d2 = h_h.shape[2]
    full = lambda *s: pl.BlockSpec(s, lambda i: (0,) * len(s))
    return pl.pallas_call(
        body,
        grid=(n // bn,),
        in_specs=[
            pl.BlockSpec((NC, 2, bn, d2), lambda i: (0, 0, i, 0)),
            pl.BlockSpec((2, bn, d2), lambda i: (0, i, 0)),
            pl.BlockSpec((bn, 1), lambda i: (i, 0)),
            full(d2, dout), full(d2, dout), full(dout),
        ],
        out_specs=pl.BlockSpec((bn, dout), lambda i: (i, 0)),
        out_shape=jax.ShapeDtypeStruct((n, dout), jnp.float32),
    )(p, h_h, dinv2_col, w_halves[0], w_halves[1], b)


def kernel(x, edge_index, edge_weight, W1, b1, W2, b2):
    n, d_in = x.shape
    e = edge_weight.shape[0]
    d_h = W1.shape[1]
    d_h_pad = ((d_h + 2 * L - 1) // (2 * L)) * (2 * L)
    da = d_in // 2
    db = d_h_pad // 2

    W1p = jnp.zeros((d_in, d_h_pad), jnp.float32).at[:, :d_h].set(W1)
    b1p = jnp.zeros((d_h_pad,), jnp.float32).at[:d_h].set(b1)
    W2p = jnp.zeros((d_h_pad, W2.shape[1]), jnp.float32).at[:d_h, :].set(W2)

    w1_q = ((W1p[:da, :db], W1p[:da, db:]),
            (W1p[da:, :db], W1p[da:, db:]))
    b1_h = jnp.stack([b1p[:db], b1p[db:]])
    w2_h = (W2p[:db], W2p[db:])

    ew_tile = e // NW
    nch = -(-ew_tile // (2 * B * C)) * 2 * B
    pad = nch * C - ew_tile
    src3 = jnp.pad(edge_index[0].reshape(NW, ew_tile), ((0, 0), (0, pad)),
                   constant_values=0).reshape(NW, nch, C)
    dst3 = jnp.pad(edge_index[1].reshape(NW, ew_tile), ((0, 0), (0, pad)),
                   constant_values=0).reshape(NW, nch, C)
    ew3 = jnp.pad(edge_weight.reshape(NW, ew_tile), ((0, 0), (0, pad)),
                  constant_values=0.0).reshape(NW, nch, C)

    x_h = jnp.stack([x[:, :da], x[:, da:]])

    deg_parts = _sc_deg(dst3, ew3, n)
    dinv, dinv2 = _tc_deg_finish(deg_parts)
    dinv2_col = dinv2.reshape(n, 1)

    p1, norm3 = _sc_agg(x_h, src3, dst3, ew3, dinv, n, compute_norm=True)
    h_h = _tc_layer1(p1, x_h, dinv2_col, w1_q, b1_h, db)
    (p2,) = _sc_agg(h_h, src3, dst3, norm3, dinv, n, compute_norm=False)
    out = _tc_layer2(p2, h_h, dinv2_col, w2_h, b2)
    return out
